# baseline (speedup 1.0000x reference)
import functools

import jax
import jax.numpy as jnp
from jax import lax
from jax.experimental import pallas as pl
from jax.experimental.pallas import tpu as pltpu
from jax.experimental.pallas import tpu_sc as plsc

N = 10000
E = 320000
H = 128
HH = H // 2

NC = 2
NS = 16
NW = NC * NS
CHUNK = 128
NCHUNK = 160
EPT = NCHUNK * CHUNK
E_PAD = NS * EPT
EPW_H = E_PAD // NW
NB = 10240
RPT = 632
OUT_ROWS = NS * RPT
TRASH = OUT_ROWS
ACC_ROWS = OUT_ROWS + 8

_ZERO16 = functools.partial(jnp.zeros, (16,), jnp.float32)


def _hist_body(dst_hbm, out_hbm, idx_v, hist_v):
    c = lax.axis_index("c")
    s = lax.axis_index("s")
    wid = s * NC + c
    pltpu.sync_copy(dst_hbm.at[wid], idx_v)

    def zero(k, _):
        hist_v[pl.ds(k * 16, 16)] = _ZERO16()
        return 0

    lax.fori_loop(0, NB // 16, zero, 0)

    ones = jnp.ones((16,), jnp.float32)

    def acc(k, _):
        idx = idx_v[pl.ds(k * 16, 16)]
        plsc.addupdate_scatter(hist_v, [idx], ones)
        return 0

    lax.fori_loop(0, EPW_H // 16, acc, 0)
    pltpu.sync_copy(hist_v, out_hbm.at[wid])


def _sc_hist(dst2):
    return pl.kernel(
        _hist_body,
        out_type=jax.ShapeDtypeStruct((NW, NB), jnp.float32),
        mesh=plsc.VectorSubcoreMesh(core_axis_name="c", subcore_axis_name="s"),
        compiler_params=pltpu.CompilerParams(needs_layout_passes=False),
        scratch_types=[
            pltpu.VMEM((EPW_H,), jnp.int32),
            pltpu.VMEM((NB,), jnp.float32),
        ],
    )(dst2)


NBUF = 2


def _gs_body(xws2_hbm, src_hbm, dst_hbm, out_hbm, src_v, dst_v, buf_v, acc_sh,
             gsems):
    c = lax.axis_index("c")
    s = lax.axis_index("s")
    pltpu.sync_copy(src_hbm.at[s], src_v)
    pltpu.sync_copy(dst_hbm.at[s], dst_v)

    def zbuf(k, _):
        buf_v[0, k // (HH // 16), pl.ds((k % (HH // 16)) * 16, 16)] = _ZERO16()
        return 0

    lax.fori_loop(0, CHUNK * HH // 16, zbuf, 0)
    for k in range(4):
        pltpu.sync_copy(buf_v.at[0], acc_sh.at[pl.ds(s * RPT + k * CHUNK, CHUNK)])
    pltpu.sync_copy(
        buf_v.at[0, pl.ds(0, RPT - 4 * CHUNK)],
        acc_sh.at[pl.ds(s * RPT + 4 * CHUNK, RPT - 4 * CHUNK)],
    )
    plsc.subcore_barrier()

    def gather(j, b):
        return pltpu.make_async_copy(
            xws2_hbm.at[c].at[src_v.at[j]], buf_v.at[b], gsems.at[b]
        )

    gather(0, 0).start()

    def group(g, _):
        for b in range(2):
            j = g * 2 + b
            nb = 1 - b

            @pl.when(j + 1 < NCHUNK)
            def _():
                gather(j + 1, nb).start()

            gather(j, b).wait()
            pltpu.sync_copy(buf_v.at[b], acc_sh.at[dst_v.at[j]], add=True)
        return 0

    lax.fori_loop(0, NCHUNK // 2, group, 0)
    plsc.subcore_barrier()
    pltpu.sync_copy(acc_sh.at[pl.ds(s * RPT, RPT)], out_hbm.at[c, pl.ds(s * RPT, RPT)])


def _sc_gather_scatter(xws2, src3, dst3):
    return pl.kernel(
        _gs_body,
        out_type=jax.ShapeDtypeStruct((NC, OUT_ROWS, HH), jnp.float32),
        mesh=plsc.VectorSubcoreMesh(core_axis_name="c", subcore_axis_name="s"),
        compiler_params=pltpu.CompilerParams(
            needs_layout_passes=False, use_tc_tiling_on_sc=False
        ),
        scratch_types=[
            pltpu.VMEM((NCHUNK, CHUNK), jnp.int32),
            pltpu.VMEM((NCHUNK, CHUNK), jnp.int32),
            pltpu.VMEM((NBUF, CHUNK, HH), jnp.float32),
            pltpu.VMEM_SHARED((ACC_ROWS, HH), jnp.float32),
            pltpu.SemaphoreType.DMA((NBUF,)),
        ],
    )(xws2, src3, dst3)


_BR = 1000


def _dense_kernel(s0_ref, s1_ref, w0_ref, b0_ref, wg_ref, hist_ref,
                  h1_ref, xws2_ref):
    h1 = jnp.dot(s0_ref[...], w0_ref[...], preferred_element_type=jnp.float32)
    h1 = h1 + b0_ref[...][None, :] + s1_ref[...]
    h1_ref[...] = h1
    xw = jnp.dot(h1, wg_ref[...], preferred_element_type=jnp.float32)
    deg = jnp.sum(hist_ref[...], axis=1) + 1.0
    dinv = lax.rsqrt(deg)
    xws = xw * dinv[:, None]
    xws2_ref[0] = xws[:, :HH]
    xws2_ref[1] = xws[:, HH:]


def _tc_dense(s0, s1, W0, b0, Wg, hist_t):
    return pl.pallas_call(
        _dense_kernel,
        grid=(N // _BR,),
        in_specs=[
            pl.BlockSpec((_BR, H), lambda i: (i, 0)),
            pl.BlockSpec((_BR, H), lambda i: (i, 0)),
            pl.BlockSpec((H, H), lambda i: (0, 0)),
            pl.BlockSpec((H,), lambda i: (0,)),
            pl.BlockSpec((H, H), lambda i: (0, 0)),
            pl.BlockSpec((_BR, NW), lambda i: (i, 0)),
        ],
        out_specs=[
            pl.BlockSpec((_BR, H), lambda i: (i, 0)),
            pl.BlockSpec((NC, _BR, HH), lambda i: (0, i, 0)),
        ],
        out_shape=[
            jax.ShapeDtypeStruct((N, H), jnp.float32),
            jax.ShapeDtypeStruct((NC, N, HH), jnp.float32),
        ],
    )(s0, s1, W0, b0, Wg, hist_t)


def _final_kernel(h1_ref, acc_ref, xws2_ref, h0_ref, bg_ref, hist_ref, out_ref):
    deg = jnp.sum(hist_ref[...], axis=1) + 1.0
    dinv = lax.rsqrt(deg)
    pre_l = dinv[:, None] * (acc_ref[0] + xws2_ref[0])
    pre_r = dinv[:, None] * (acc_ref[1] + xws2_ref[1])
    pre = jnp.concatenate([pre_l, pre_r], axis=1)
    pre = pre + bg_ref[...][None, :] + h0_ref[...]
    out_ref[...] = h1_ref[...] + jnp.maximum(pre, 0.0)


def _tc_final(h1, accs, xws2, h0, bg, hist_t):
    return pl.pallas_call(
        _final_kernel,
        grid=(N // _BR,),
        in_specs=[
            pl.BlockSpec((_BR, H), lambda i: (i, 0)),
            pl.BlockSpec((NC, _BR, HH), lambda i: (0, i, 0)),
            pl.BlockSpec((NC, _BR, HH), lambda i: (0, i, 0)),
            pl.BlockSpec((_BR, H), lambda i: (i, 0)),
            pl.BlockSpec((H,), lambda i: (0,)),
            pl.BlockSpec((_BR, NW), lambda i: (i, 0)),
        ],
        out_specs=pl.BlockSpec((_BR, H), lambda i: (i, 0)),
        out_shape=jax.ShapeDtypeStruct((N, H), jnp.float32),
    )(h1, accs, xws2, h0, bg, hist_t)


def kernel(s0, s1, edge_index, h0, W0, b0, Wg, bg):
    src = edge_index[0].astype(jnp.int32)
    dst = edge_index[1].astype(jnp.int32)
    pad = E_PAD - E
    srcp = jnp.concatenate([src, jnp.zeros((pad,), jnp.int32)])
    dstp = jnp.concatenate([dst, jnp.full((pad,), TRASH, jnp.int32)])
    src3 = srcp.reshape(NS, NCHUNK, CHUNK)
    dst3 = dstp.reshape(NS, NCHUNK, CHUNK)
    dst2 = dstp.reshape(NW, EPW_H)

    hist = _sc_hist(dst2)
    hist_t = hist.T
    h1, xws2 = _tc_dense(s0, s1, W0, b0, Wg, hist_t)
    accs = _sc_gather_scatter(xws2, src3, dst3)
    return _tc_final(h1, accs, xws2, h0, bg, hist_t)


# speedup vs baseline: 1.3059x; 1.3059x over previous
import functools

import jax
import jax.numpy as jnp
from jax import lax
from jax.experimental import pallas as pl
from jax.experimental.pallas import tpu as pltpu
from jax.experimental.pallas import tpu_sc as plsc

N = 10000
E = 320000
H = 128
HH = H // 2

NC = 2
NS = 16
NW = NC * NS
CHUNK = 125
NCHUNK = 160
EPT = NCHUNK * CHUNK
EPW_H = E // NW
NB = 10112
RPT = 632
OUT_ROWS = NS * RPT
ACC_ROWS = OUT_ROWS + 8

_ZERO16 = functools.partial(jnp.zeros, (16,), jnp.float32)


def _hist_body(dst_hbm, out_hbm, idx_v, hist_v):
    c = lax.axis_index("c")
    s = lax.axis_index("s")
    wid = s * NC + c
    pltpu.sync_copy(dst_hbm.at[wid], idx_v)

    def zero(k, _):
        hist_v[pl.ds(k * 16, 16)] = _ZERO16()
        return 0

    lax.fori_loop(0, NB // 16, zero, 0)

    ones = jnp.ones((16,), jnp.float32)

    def acc(k, _):
        idx = idx_v[pl.ds(k * 16, 16)]
        plsc.addupdate_scatter(hist_v, [idx], ones)
        return 0

    lax.fori_loop(0, EPW_H // 16, acc, 0)
    pltpu.sync_copy(hist_v, out_hbm.at[wid])


def _sc_hist(dst2):
    return pl.kernel(
        _hist_body,
        out_type=jax.ShapeDtypeStruct((NW, NB), jnp.float32),
        mesh=plsc.VectorSubcoreMesh(core_axis_name="c", subcore_axis_name="s"),
        compiler_params=pltpu.CompilerParams(needs_layout_passes=False),
        scratch_types=[
            pltpu.VMEM((EPW_H,), jnp.int32),
            pltpu.VMEM((NB,), jnp.float32),
        ],
    )(dst2)


def _gs_body(xws2_hbm, edges_hbm, out_hbm, src_v, dst_v, buf_v, acc_sh):
    c = lax.axis_index("c")
    s = lax.axis_index("s")
    pltpu.sync_copy(edges_hbm.at[0, pl.ds(s * NCHUNK, NCHUNK)], src_v)
    pltpu.sync_copy(edges_hbm.at[1, pl.ds(s * NCHUNK, NCHUNK)], dst_v)

    def zbuf(k, _):
        buf_v[k // (HH // 16), pl.ds((k % (HH // 16)) * 16, 16)] = _ZERO16()
        return 0

    lax.fori_loop(0, 128 * HH // 16, zbuf, 0)
    for k in range(4):
        pltpu.sync_copy(buf_v.at[pl.ds(0, 128)],
                        acc_sh.at[pl.ds(s * RPT + k * 128, 128)])
    pltpu.sync_copy(
        buf_v.at[pl.ds(0, RPT - 4 * 128)],
        acc_sh.at[pl.ds(s * RPT + 4 * 128, RPT - 4 * 128)],
    )
    plsc.subcore_barrier()

    def step(j, _):
        pltpu.sync_copy(xws2_hbm.at[c].at[src_v.at[j]], buf_v.at[pl.ds(0, CHUNK)])
        pltpu.sync_copy(buf_v.at[pl.ds(0, CHUNK)], acc_sh.at[dst_v.at[j]], add=True)
        return 0

    lax.fori_loop(0, NCHUNK, step, 0)
    plsc.subcore_barrier()
    pltpu.sync_copy(acc_sh.at[pl.ds(s * RPT, RPT)], out_hbm.at[c, pl.ds(s * RPT, RPT)])


def _sc_gather_scatter(xws2, edges3):
    return pl.kernel(
        _gs_body,
        out_type=jax.ShapeDtypeStruct((NC, OUT_ROWS, HH), jnp.float32),
        mesh=plsc.VectorSubcoreMesh(core_axis_name="c", subcore_axis_name="s"),
        compiler_params=pltpu.CompilerParams(
            needs_layout_passes=False, use_tc_tiling_on_sc=False
        ),
        scratch_types=[
            pltpu.VMEM((NCHUNK, CHUNK), jnp.int32),
            pltpu.VMEM((NCHUNK, CHUNK), jnp.int32),
            pltpu.VMEM((128, HH), jnp.float32),
            pltpu.VMEM_SHARED((ACC_ROWS, HH), jnp.float32),
        ],
    )(xws2, edges3)


_BR = 1000


def _dense_kernel(s0_ref, s1_ref, w0_ref, b0_ref, wg_ref, hist_ref,
                  h1_ref, xws2_ref):
    h1 = jnp.dot(s0_ref[...], w0_ref[...], preferred_element_type=jnp.float32)
    h1 = h1 + b0_ref[...][None, :] + s1_ref[...]
    h1_ref[...] = h1
    xw = jnp.dot(h1, wg_ref[...], preferred_element_type=jnp.float32)
    deg = jnp.sum(hist_ref[...], axis=1) + 1.0
    dinv = lax.rsqrt(deg)
    xws = xw * dinv[:, None]
    xws2_ref[0] = xws[:, :HH]
    xws2_ref[1] = xws[:, HH:]


def _tc_dense(s0, s1, W0, b0, Wg, hist_t):
    return pl.pallas_call(
        _dense_kernel,
        grid=(N // _BR,),
        in_specs=[
            pl.BlockSpec((_BR, H), lambda i: (i, 0)),
            pl.BlockSpec((_BR, H), lambda i: (i, 0)),
            pl.BlockSpec((H, H), lambda i: (0, 0)),
            pl.BlockSpec((H,), lambda i: (0,)),
            pl.BlockSpec((H, H), lambda i: (0, 0)),
            pl.BlockSpec((_BR, NW), lambda i: (i, 0)),
        ],
        out_specs=[
            pl.BlockSpec((_BR, H), lambda i: (i, 0)),
            pl.BlockSpec((NC, _BR, HH), lambda i: (0, i, 0)),
        ],
        out_shape=[
            jax.ShapeDtypeStruct((N, H), jnp.float32),
            jax.ShapeDtypeStruct((NC, N, HH), jnp.float32),
        ],
    )(s0, s1, W0, b0, Wg, hist_t)


def _final_kernel(h1_ref, acc_ref, xws2_ref, h0_ref, bg_ref, hist_ref, out_ref):
    deg = jnp.sum(hist_ref[...], axis=1) + 1.0
    dinv = lax.rsqrt(deg)
    pre_l = dinv[:, None] * (acc_ref[0] + xws2_ref[0])
    pre_r = dinv[:, None] * (acc_ref[1] + xws2_ref[1])
    pre = jnp.concatenate([pre_l, pre_r], axis=1)
    pre = pre + bg_ref[...][None, :] + h0_ref[...]
    out_ref[...] = h1_ref[...] + jnp.maximum(pre, 0.0)


def _tc_final(h1, accs, xws2, h0, bg, hist_t):
    return pl.pallas_call(
        _final_kernel,
        grid=(N // _BR,),
        in_specs=[
            pl.BlockSpec((_BR, H), lambda i: (i, 0)),
            pl.BlockSpec((NC, _BR, HH), lambda i: (0, i, 0)),
            pl.BlockSpec((NC, _BR, HH), lambda i: (0, i, 0)),
            pl.BlockSpec((_BR, H), lambda i: (i, 0)),
            pl.BlockSpec((H,), lambda i: (0,)),
            pl.BlockSpec((_BR, NW), lambda i: (i, 0)),
        ],
        out_specs=pl.BlockSpec((_BR, H), lambda i: (i, 0)),
        out_shape=jax.ShapeDtypeStruct((N, H), jnp.float32),
    )(h1, accs, xws2, h0, bg, hist_t)


def kernel(s0, s1, edge_index, h0, W0, b0, Wg, bg):
    edges3 = edge_index.astype(jnp.int32).reshape(2, NS * NCHUNK, CHUNK)
    dst2 = edge_index[1].astype(jnp.int32).reshape(NW, EPW_H)

    hist = _sc_hist(dst2)
    hist_t = hist.T
    h1, xws2 = _tc_dense(s0, s1, W0, b0, Wg, hist_t)
    accs = _sc_gather_scatter(xws2, edges3)
    return _tc_final(h1, accs, xws2, h0, bg, hist_t)


# speedup vs baseline: 2.6955x; 2.0641x over previous
import functools

import jax
import jax.numpy as jnp
from jax import lax
from jax.experimental import pallas as pl
from jax.experimental.pallas import tpu as pltpu
from jax.experimental.pallas import tpu_sc as plsc

N = 10000
E = 320000
H = 128

NC = 2
NS = 16
NW = NC * NS
CHUNK = 125
NCHUNK = 160
EPW_H = E // NW
NB = 10112
RPT = 632
OUT_ROWS = NS * RPT
ACC_ROWS = OUT_ROWS + 8

_ZERO16 = functools.partial(jnp.zeros, (16,), jnp.float32)


def _hist_body(dst_hbm, out_hbm, idx_v, hist_v):
    c = lax.axis_index("c")
    s = lax.axis_index("s")
    wid = s * NC + c
    pltpu.sync_copy(dst_hbm.at[wid], idx_v)

    def zero(k, _):
        hist_v[pl.ds(k * 16, 16)] = _ZERO16()
        return 0

    lax.fori_loop(0, NB // 16, zero, 0)

    ones = jnp.ones((16,), jnp.float32)

    def acc(k, _):
        idx = idx_v[pl.ds(k * 16, 16)]
        plsc.addupdate_scatter(hist_v, [idx], ones)
        return 0

    lax.fori_loop(0, EPW_H // 16, acc, 0)
    pltpu.sync_copy(hist_v, out_hbm.at[wid])


def _sc_hist(dst2):
    return pl.kernel(
        _hist_body,
        out_type=jax.ShapeDtypeStruct((NW, NB), jnp.float32),
        mesh=plsc.VectorSubcoreMesh(core_axis_name="c", subcore_axis_name="s"),
        compiler_params=pltpu.CompilerParams(needs_layout_passes=False),
        scratch_types=[
            pltpu.VMEM((EPW_H,), jnp.int32),
            pltpu.VMEM((NB,), jnp.float32),
        ],
    )(dst2)


NCHUNK_W = NCHUNK // NC


def _gs_body(xws_hbm, edges_hbm, out_hbm, src_v, dst_v, buf_v, gbuf_v, acc_sh,
             gsem, ssem):
    c = lax.axis_index("c")
    s = lax.axis_index("s")
    w = s * NC + c
    pltpu.sync_copy(edges_hbm.at[0, pl.ds(w * NCHUNK_W, NCHUNK_W)], src_v)
    pltpu.sync_copy(edges_hbm.at[1, pl.ds(w * NCHUNK_W, NCHUNK_W)], dst_v)

    def zbuf(k, _):
        buf_v[k // (H // 32), pl.ds((k % (H // 32)) * 32, 32)] = jnp.zeros(
            (32,), jnp.bfloat16
        )
        return 0

    lax.fori_loop(0, 128 * H // 32, zbuf, 0)
    for k in range(4):
        pltpu.sync_copy(buf_v.at[pl.ds(0, 128)],
                        acc_sh.at[pl.ds(s * RPT + k * 128, 128)])
    pltpu.sync_copy(
        buf_v.at[pl.ds(0, RPT - 4 * 128)],
        acc_sh.at[pl.ds(s * RPT + 4 * 128, RPT - 4 * 128)],
    )
    plsc.subcore_barrier()

    def g_desc(j, slot):
        return pltpu.make_async_copy(
            xws_hbm.at[src_v.at[j]], gbuf_v.at[slot], gsem
        )

    def s_desc(j, slot):
        return pltpu.make_async_copy(
            gbuf_v.at[slot], acc_sh.at[dst_v.at[j]], ssem
        )

    def fire_g(i, half):
        for t in range(K):
            g_desc(i * K + t, half * K + t).start()

    def drain_g(i, half):
        for t in range(K):
            g_desc(i * K + t, half * K + t).wait()

    def fire_s(i, half):
        for t in range(K):
            pltpu.async_copy(
                gbuf_v.at[half * K + t],
                acc_sh.at[dst_v.at[i * K + t]],
                ssem,
                add=True,
            )

    def drain_s(i, half):
        for t in range(K):
            s_desc(i * K + t, half * K + t).wait()

    fire_g(0, 0)

    def group(g, _):
        for p in range(2):
            i = g * 2 + p
            drain_g(i, p)

            @pl.when(i + 1 < NBATCH)
            def _():
                @pl.when(i >= 1)
                def _():
                    drain_s(i - 1, 1 - p)

                fire_g(i + 1, 1 - p)

            fire_s(i, p)
        return 0

    lax.fori_loop(0, NBATCH // 2, group, 0)
    drain_s(NBATCH - 2, 0 if (NBATCH - 2) % 2 == 0 else 1)
    drain_s(NBATCH - 1, (NBATCH - 1) % 2)
    plsc.subcore_barrier()
    pltpu.sync_copy(acc_sh.at[pl.ds(s * RPT, RPT)], out_hbm.at[c, pl.ds(s * RPT, RPT)])


K = 2
NBATCH = NCHUNK_W // K


def _sc_gather_scatter(xws_bf, edges3):
    return pl.kernel(
        _gs_body,
        out_type=jax.ShapeDtypeStruct((NC, OUT_ROWS, H), jnp.bfloat16),
        mesh=plsc.VectorSubcoreMesh(core_axis_name="c", subcore_axis_name="s"),
        compiler_params=pltpu.CompilerParams(
            needs_layout_passes=False, use_tc_tiling_on_sc=False
        ),
        scratch_types=[
            pltpu.VMEM((NCHUNK_W, CHUNK), jnp.int32),
            pltpu.VMEM((NCHUNK_W, CHUNK), jnp.int32),
            pltpu.VMEM((128, H), jnp.bfloat16),
            pltpu.VMEM((2 * K, CHUNK, H), jnp.bfloat16),
            pltpu.VMEM_SHARED((ACC_ROWS, H), jnp.bfloat16),
            pltpu.SemaphoreType.DMA,
            pltpu.SemaphoreType.DMA,
        ],
    )(xws_bf, edges3)


def _deg_inv(hist_ref):
    deg = jnp.sum(hist_ref[:, :N], axis=0) + 1.0
    return lax.rsqrt(deg)


def _mm_kernel(s0_ref, s1_ref, w0_ref, b0_ref, wg_ref, h1_ref, xw_ref):
    h1 = jnp.dot(s0_ref[...], w0_ref[...], preferred_element_type=jnp.float32)
    h1 = h1 + b0_ref[...][None, :] + s1_ref[...]
    h1_ref[...] = h1.astype(jnp.bfloat16)
    xw_ref[...] = jnp.dot(h1, wg_ref[...], preferred_element_type=jnp.float32)


def _tc_mm(s0, s1, W0, b0, Wg):
    return pl.pallas_call(
        _mm_kernel,
        out_shape=[
            jax.ShapeDtypeStruct((N, H), jnp.bfloat16),
            jax.ShapeDtypeStruct((N, H), jnp.float32),
        ],
    )(s0, s1, W0, b0, Wg)


def _scale_kernel(xw_ref, hist_ref, xws_ref):
    dinv = _deg_inv(hist_ref)
    xws_ref[...] = (xw_ref[...] * dinv[:, None]).astype(jnp.bfloat16)


def _tc_scale(xw, hist):
    return pl.pallas_call(
        _scale_kernel,
        out_shape=jax.ShapeDtypeStruct((N, H), jnp.bfloat16),
    )(xw, hist)


def _final_kernel(h1_ref, acc_ref, xws_ref, h0_ref, bg_ref, hist_ref, out_ref):
    dinv = _deg_inv(hist_ref)
    acc = (acc_ref[0, :N].astype(jnp.float32) + acc_ref[1, :N].astype(jnp.float32)
           + xws_ref[...].astype(jnp.float32))
    pre = dinv[:, None] * acc + bg_ref[...][None, :] + h0_ref[...]
    out_ref[...] = h1_ref[...].astype(jnp.float32) + jnp.maximum(pre, 0.0)


def _tc_final(h1, accs, xws_bf, h0, bg, hist):
    return pl.pallas_call(
        _final_kernel,
        out_shape=jax.ShapeDtypeStruct((N, H), jnp.float32),
    )(h1, accs, xws_bf, h0, bg, hist)


def kernel(s0, s1, edge_index, h0, W0, b0, Wg, bg):
    edges3 = edge_index.astype(jnp.int32).reshape(2, NS * NCHUNK, CHUNK)
    dst2 = edges3[1].reshape(NW, EPW_H)

    hist = _sc_hist(dst2)
    h1, xw = _tc_mm(s0, s1, W0, b0, Wg)
    xws_bf = _tc_scale(xw, hist)
    accs = _sc_gather_scatter(xws_bf, edges3)
    return _tc_final(h1, accs, xws_bf, h0, bg, hist)
